# baseline (device time: 81621 ns/iter reference)
import jax
import jax.numpy as jnp
from jax import lax
from jax.experimental import pallas as pl
from jax.experimental.pallas import tpu as pltpu

CHUNK = 32
CLS = 896
NCC = CLS // CHUNK
XSTART = 4 * CLS
NXE = 512 // CHUNK
NXF = NCC + NXE
NHA = 14
NHB = 14
NYF = NCC + NHA
NZF = NCC + NHB


def kernel(x, pi):
    _, m, n = x.shape
    n_tot = m // CHUNK

    def body(
        x_ref,
        pi_ref,
        out_ref,
        loadbuf,
        sendbuf,
        rbx,
        rby,
        rbz,
        ld_sems,
        sx,
        rx,
        sy,
        ry,
        sz,
        rz,
        st,
    ):
        mx = lax.axis_index("x")
        my = lax.axis_index("y")
        mz = lax.axis_index("z")
        zbit = mz % 2
        xn = (1 - mx, my, mz)
        yn = (mx, 1 - my, mz)
        zn = (mx, my, mz + 1 - 2 * zbit)

        c_me = 2 * my + zbit
        c_yn = 2 * (1 - my) + zbit
        c_zn = 2 * my + (1 - zbit)
        c_dg = 2 * (1 - my) + (1 - zbit)

        swap = pi_ref[mx] != mx

        def load(row_start, slot):
            return pltpu.make_async_copy(
                x_ref.at[0, pl.ds(row_start, CHUNK), :],
                loadbuf.at[slot],
                ld_sems.at[slot],
            )

        def rdma(src, dst, ssem, rsem, dev):
            return pltpu.make_async_remote_copy(
                src_ref=src,
                dst_ref=dst,
                send_sem=ssem,
                recv_sem=rsem,
                device_id=dev,
                device_id_type=pl.DeviceIdType.MESH,
            )

        def store(src, row_start, sem):
            return pltpu.make_async_copy(
                src, out_ref.at[0, pl.ds(row_start, CHUNK), :], sem
            )

        def send_row(i):
            if i < NCC:
                return c_me * CLS + i * CHUNK
            return XSTART + (i - NCC) * CHUNK

        def x_flow(i):
            if i < NCC:
                dst = rbx.at[i]
            else:
                dst = out_ref.at[0, pl.ds(XSTART + (i - NCC) * CHUNK, CHUNK), :]
            return rdma(sendbuf.at[i], dst, sx.at[i], rx.at[i], xn)

        def y_flow(i):
            src = rbx.at[i] if i < NCC else rbz.at[i - NCC]
            return rdma(src, rby.at[i], sy.at[i], ry.at[i], yn)

        def z_flow(i):
            src = rbx.at[i] if i < NCC else rby.at[NHA + i - NCC]
            return rdma(src, rbz.at[i], sz.at[i], rz.at[i], zn)

        for i in range(NXF):

            @pl.when(swap)
            def _(i=i):
                load(send_row(i), i).start()

            @pl.when(jnp.logical_not(swap))
            def _(i=i):
                load(i * CHUNK, i).start()

        barrier = pltpu.get_barrier_semaphore()
        for nbr in (xn, yn, zn):
            pl.semaphore_signal(
                barrier, inc=1, device_id=nbr, device_id_type=pl.DeviceIdType.MESH
            )
        pl.semaphore_wait(barrier, 3)

        @pl.when(swap)
        def _():
            stores = []

            def start_store(src, row_start):
                sem_idx = len(stores)
                stores.append((src, row_start, sem_idx))
                store(src, row_start, st.at[sem_idx]).start()

            for i in range(NXF):
                load(send_row(i), i).wait()
                sendbuf[i] = loadbuf[i].astype(jnp.bfloat16)
                x_flow(i).start()

            for i in range(NCC):
                x_flow(i).wait_recv()
                y_flow(i).start()
                z_flow(i).start()
                start_store(rbx.at[i], c_me * CLS + i * CHUNK)

            for i in range(NHA):
                z_flow(i).wait_recv()
                y_flow(NCC + i).start()
                start_store(rbz.at[i], c_zn * CLS + i * CHUNK)

            for i in range(NHA, NCC):
                y_flow(i).wait_recv()
                z_flow(NCC + i - NHA).start()
                start_store(rby.at[i], c_yn * CLS + i * CHUNK)

            for i in range(NHA):
                y_flow(i).wait_recv()
                start_store(rby.at[i], c_yn * CLS + i * CHUNK)
            for i in range(NHA, NCC):
                z_flow(i).wait_recv()
                start_store(rbz.at[i], c_zn * CLS + i * CHUNK)
            for i in range(NCC, NYF):
                y_flow(i).wait_recv()
                start_store(rby.at[i], c_dg * CLS + (i - NCC) * CHUNK)
            for i in range(NCC, NZF):
                z_flow(i).wait_recv()
                start_store(rbz.at[i], c_dg * CLS + NHA * CHUNK + (i - NCC) * CHUNK)
            for i in range(NCC, NXF):
                x_flow(i).wait_recv()

            for i in range(NXF):
                x_flow(i).wait_send()
            for i in range(NYF):
                y_flow(i).wait_send()
            for i in range(NZF):
                z_flow(i).wait_send()
            for src, row_start, sem_idx in stores:
                store(src, row_start, st.at[sem_idx]).wait()

        @pl.when(jnp.logical_not(swap))
        def _():
            for k in range(n_tot):
                load(k * CHUNK, k % NXF).wait()
                if k >= NXF:
                    store(
                        sendbuf.at[(k - NXF) % NXF],
                        (k - NXF) * CHUNK,
                        st.at[k - NXF],
                    ).wait()
                sendbuf[k % NXF] = loadbuf[k % NXF].astype(jnp.bfloat16)
                store(sendbuf.at[k % NXF], k * CHUNK, st.at[k]).start()
                if k + NXF < n_tot:
                    load((k + NXF) * CHUNK, k % NXF).start()
            for k in range(n_tot - NXF, n_tot):
                store(sendbuf.at[k % NXF], k * CHUNK, st.at[k]).wait()

    return pl.pallas_call(
        body,
        out_shape=jax.ShapeDtypeStruct(x.shape, jnp.bfloat16),
        in_specs=[
            pl.BlockSpec(memory_space=pl.ANY),
            pl.BlockSpec(memory_space=pltpu.SMEM),
        ],
        out_specs=pl.BlockSpec(memory_space=pl.ANY),
        scratch_shapes=[
            pltpu.VMEM((NXF, CHUNK, n), jnp.float32),
            pltpu.VMEM((NXF, CHUNK, n), jnp.bfloat16),
            pltpu.VMEM((NCC, CHUNK, n), jnp.bfloat16),
            pltpu.VMEM((NYF, CHUNK, n), jnp.bfloat16),
            pltpu.VMEM((NZF, CHUNK, n), jnp.bfloat16),
            pltpu.SemaphoreType.DMA((NXF,)),
            pltpu.SemaphoreType.DMA((NXF,)),
            pltpu.SemaphoreType.DMA((NXF,)),
            pltpu.SemaphoreType.DMA((NYF,)),
            pltpu.SemaphoreType.DMA((NYF,)),
            pltpu.SemaphoreType.DMA((NZF,)),
            pltpu.SemaphoreType.DMA((NZF,)),
            pltpu.SemaphoreType.DMA((n_tot,)),
        ],
        compiler_params=pltpu.CompilerParams(collective_id=0),
    )(x, pi)


# device time: 78550 ns/iter; 1.0391x vs baseline; 1.0391x over previous
import jax
import jax.numpy as jnp
from jax import lax
from jax.experimental import pallas as pl
from jax.experimental.pallas import tpu as pltpu

CHUNK = 64
CLS = 896
NCC = CLS // CHUNK
XSTART = 4 * CLS
NXE = 512 // CHUNK
NXF = NCC + NXE
NHA = 7
NHB = 7
NYF = NCC + NHA
NZF = NCC + NHB


def kernel(x, pi):
    _, m, n = x.shape
    n_tot = m // CHUNK

    def body(
        x_ref,
        pi_ref,
        out_ref,
        loadbuf,
        sendbuf,
        rbx,
        rby,
        rbz,
        ld_sems,
        sx,
        rx,
        sy,
        ry,
        sz,
        rz,
        st,
    ):
        mx = lax.axis_index("x")
        my = lax.axis_index("y")
        mz = lax.axis_index("z")
        zbit = mz % 2
        xn = (1 - mx, my, mz)
        yn = (mx, 1 - my, mz)
        zn = (mx, my, mz + 1 - 2 * zbit)

        c_me = 2 * my + zbit
        c_yn = 2 * (1 - my) + zbit
        c_zn = 2 * my + (1 - zbit)
        c_dg = 2 * (1 - my) + (1 - zbit)

        swap = pi_ref[mx] != mx

        def load(row_start, slot):
            return pltpu.make_async_copy(
                x_ref.at[0, pl.ds(row_start, CHUNK), :],
                loadbuf.at[slot],
                ld_sems.at[slot],
            )

        def rdma(src, dst, ssem, rsem, dev):
            return pltpu.make_async_remote_copy(
                src_ref=src,
                dst_ref=dst,
                send_sem=ssem,
                recv_sem=rsem,
                device_id=dev,
                device_id_type=pl.DeviceIdType.MESH,
            )

        def store(src, row_start, sem):
            return pltpu.make_async_copy(
                src, out_ref.at[0, pl.ds(row_start, CHUNK), :], sem
            )

        def send_row(i):
            if i < NCC:
                return c_me * CLS + i * CHUNK
            return XSTART + (i - NCC) * CHUNK

        def x_flow(i):
            if i < NCC:
                dst = rbx.at[i]
            else:
                dst = out_ref.at[0, pl.ds(XSTART + (i - NCC) * CHUNK, CHUNK), :]
            return rdma(sendbuf.at[i], dst, sx.at[i], rx.at[i], xn)

        def y_flow(i):
            src = rbx.at[i] if i < NCC else rbz.at[i - NCC]
            return rdma(src, rby.at[i], sy.at[i], ry.at[i], yn)

        def z_flow(i):
            src = rbx.at[i] if i < NCC else rby.at[NHA + i - NCC]
            return rdma(src, rbz.at[i], sz.at[i], rz.at[i], zn)

        for i in range(NXF):

            @pl.when(swap)
            def _(i=i):
                load(send_row(i), i).start()

            @pl.when(jnp.logical_not(swap))
            def _(i=i):
                load(i * CHUNK, i).start()

        barrier = pltpu.get_barrier_semaphore()
        for nbr in (xn, yn, zn):
            pl.semaphore_signal(
                barrier, inc=1, device_id=nbr, device_id_type=pl.DeviceIdType.MESH
            )
        pl.semaphore_wait(barrier, 3)

        @pl.when(swap)
        def _():
            stores = []

            def start_store(src, row_start):
                sem_idx = len(stores)
                stores.append((src, row_start, sem_idx))
                store(src, row_start, st.at[sem_idx]).start()

            for i in range(NXF):
                load(send_row(i), i).wait()
                sendbuf[i] = loadbuf[i].astype(jnp.bfloat16)
                x_flow(i).start()

            for i in range(NCC):
                x_flow(i).wait_recv()
                y_flow(i).start()
                z_flow(i).start()
                start_store(rbx.at[i], c_me * CLS + i * CHUNK)

            for i in range(NHA):
                z_flow(i).wait_recv()
                y_flow(NCC + i).start()
                start_store(rbz.at[i], c_zn * CLS + i * CHUNK)

            for i in range(NHA, NCC):
                y_flow(i).wait_recv()
                z_flow(NCC + i - NHA).start()
                start_store(rby.at[i], c_yn * CLS + i * CHUNK)

            for i in range(NHA):
                y_flow(i).wait_recv()
                start_store(rby.at[i], c_yn * CLS + i * CHUNK)
            for i in range(NHA, NCC):
                z_flow(i).wait_recv()
                start_store(rbz.at[i], c_zn * CLS + i * CHUNK)
            for i in range(NCC, NYF):
                y_flow(i).wait_recv()
                start_store(rby.at[i], c_dg * CLS + (i - NCC) * CHUNK)
            for i in range(NCC, NZF):
                z_flow(i).wait_recv()
                start_store(rbz.at[i], c_dg * CLS + NHA * CHUNK + (i - NCC) * CHUNK)
            for i in range(NCC, NXF):
                x_flow(i).wait_recv()

            for i in range(NXF):
                x_flow(i).wait_send()
            for i in range(NYF):
                y_flow(i).wait_send()
            for i in range(NZF):
                z_flow(i).wait_send()
            for src, row_start, sem_idx in stores:
                store(src, row_start, st.at[sem_idx]).wait()

        @pl.when(jnp.logical_not(swap))
        def _():
            for k in range(n_tot):
                load(k * CHUNK, k % NXF).wait()
                if k >= NXF:
                    store(
                        sendbuf.at[(k - NXF) % NXF],
                        (k - NXF) * CHUNK,
                        st.at[k - NXF],
                    ).wait()
                sendbuf[k % NXF] = loadbuf[k % NXF].astype(jnp.bfloat16)
                store(sendbuf.at[k % NXF], k * CHUNK, st.at[k]).start()
                if k + NXF < n_tot:
                    load((k + NXF) * CHUNK, k % NXF).start()
            for k in range(n_tot - NXF, n_tot):
                store(sendbuf.at[k % NXF], k * CHUNK, st.at[k]).wait()

    return pl.pallas_call(
        body,
        out_shape=jax.ShapeDtypeStruct(x.shape, jnp.bfloat16),
        in_specs=[
            pl.BlockSpec(memory_space=pl.ANY),
            pl.BlockSpec(memory_space=pltpu.SMEM),
        ],
        out_specs=pl.BlockSpec(memory_space=pl.ANY),
        scratch_shapes=[
            pltpu.VMEM((NXF, CHUNK, n), jnp.float32),
            pltpu.VMEM((NXF, CHUNK, n), jnp.bfloat16),
            pltpu.VMEM((NCC, CHUNK, n), jnp.bfloat16),
            pltpu.VMEM((NYF, CHUNK, n), jnp.bfloat16),
            pltpu.VMEM((NZF, CHUNK, n), jnp.bfloat16),
            pltpu.SemaphoreType.DMA((NXF,)),
            pltpu.SemaphoreType.DMA((NXF,)),
            pltpu.SemaphoreType.DMA((NXF,)),
            pltpu.SemaphoreType.DMA((NYF,)),
            pltpu.SemaphoreType.DMA((NYF,)),
            pltpu.SemaphoreType.DMA((NZF,)),
            pltpu.SemaphoreType.DMA((NZF,)),
            pltpu.SemaphoreType.DMA((n_tot,)),
        ],
        compiler_params=pltpu.CompilerParams(collective_id=0),
    )(x, pi)
